# Initial kernel scaffold; baseline (speedup 1.0000x reference)
import jax
import jax.numpy as jnp
from jax.experimental import pallas as pl


def kernel(X, center):
    raise NotImplementedError("write your pallas kernel here")


# speedup vs baseline: 1.1165x; 1.1165x over previous
import functools

import jax
import jax.numpy as jnp
from jax import lax
from jax.experimental import pallas as pl
from jax.experimental.pallas import tpu as pltpu
from jax.experimental.pallas import tpu_sc as plsc

_M = 8
_K = 256
_D = 256
_DSUB = _D // _M
_BLK = 512

_SC_NC = 2
_SC_NS = 16
_SC_NW = _SC_NC * _SC_NS


def _assign_body(x_ref, cen_ref, lab_ref, idx_ref, loss_ref):
    m = pl.program_id(0)
    x = x_ref[0]
    cm = cen_ref[0]
    csq = jnp.sum(cm * cm, axis=1, keepdims=True)
    xsq = jnp.sum(x * x, axis=0, keepdims=True)
    scores = lax.dot_general(
        cm, x, (((1,), (0,)), ((), ())),
        preferred_element_type=jnp.float32)
    adj2 = (csq - 2.0 * scores) + xsq
    dist = -jnp.sqrt(adj2)
    maxv = jnp.max(dist, axis=0, keepdims=True)
    kiota = lax.broadcasted_iota(jnp.int32, dist.shape, 0)
    lab = jnp.min(jnp.where(dist == maxv, kiota, _K),
                  axis=0, keepdims=True)
    lab_ref[...] = lab[None]
    idx_ref[...] = lab[None] + m * _K
    partial = jnp.sum(jnp.min(adj2, axis=0))
    loss_ref[...] = jnp.full((1, 1, 8, 128), partial, jnp.float32)


def _assign(X1, center):
    B = X1.shape[2]
    nb = B // _BLK
    return pl.pallas_call(
        _assign_body,
        grid=(_M, nb),
        in_specs=[
            pl.BlockSpec((1, _DSUB, _BLK), lambda m, i: (m, 0, i)),
            pl.BlockSpec((1, _K, _DSUB), lambda m, i: (m, 0, 0)),
        ],
        out_specs=[
            pl.BlockSpec((1, 1, _BLK), lambda m, i: (m, 0, i)),
            pl.BlockSpec((1, 1, _BLK), lambda m, i: (m, 0, i)),
            pl.BlockSpec((1, 1, 8, 128), lambda m, i: (m, i, 0, 0)),
        ],
        out_shape=[
            jax.ShapeDtypeStruct((_M, 1, B), jnp.int32),
            jax.ShapeDtypeStruct((_M, 1, B), jnp.int32),
            jax.ShapeDtypeStruct((_M, nb, 8, 128), jnp.float32),
        ],
    )(X1, center)


def _sc_gather(table, idx):
    n = idx.shape[0]
    bpw = n // _SC_NW

    @functools.partial(
        pl.kernel,
        mesh=plsc.VectorSubcoreMesh(core_axis_name="c", subcore_axis_name="s"),
        out_type=jax.ShapeDtypeStruct((n, _DSUB), jnp.float32),
        scratch_types=[
            pltpu.VMEM((bpw,), jnp.int32),
            pltpu.VMEM((bpw, _DSUB), jnp.float32),
            pltpu.SemaphoreType.DMA,
        ],
        compiler_params=pltpu.CompilerParams(use_tc_tiling_on_sc=False),
    )
    def gk(table_hbm, idx_hbm, out_hbm, idx_v, rows_v, sem):
        wid = lax.axis_index("s") * _SC_NC + lax.axis_index("c")
        base = wid * bpw
        pltpu.sync_copy(idx_hbm.at[pl.ds(base, bpw)], idx_v)
        pltpu.async_copy(table_hbm.at[idx_v], rows_v, sem).wait()
        pltpu.sync_copy(rows_v, out_hbm.at[pl.ds(base, bpw)])

    return gk(table, idx)


def kernel(X, center):
    B = X.shape[0]
    X1 = jnp.transpose(X.reshape(B, _M, _DSUB), (1, 2, 0))
    lab3, idx3, lossp = _assign(X1, center)
    idx = jnp.swapaxes(idx3[:, 0, :], 0, 1).reshape(B * _M)
    rows = _sc_gather(center.reshape(_M * _K, _DSUB), idx)
    X_r_out = rows.reshape(B, _M, _DSUB)
    X_r_m = rows.reshape(B, _D)
    X_p = X.reshape(B, _M, _DSUB)
    label = jnp.swapaxes(lab3[:, 0, :], 0, 1)[..., None]
    loss = jnp.sum(lossp[:, :, 0, 0]) * jnp.float32(2.0 / (B * _D))
    return (X_r_out, X_p, X_r_m, X, center, label, loss)
